# baseline (speedup 1.0000x reference)
import functools

import jax
import jax.numpy as jnp
from jax import lax
from jax.experimental import pallas as pl
from jax.experimental.pallas import tpu as pltpu
from jax.experimental.pallas import tpu_sc as plsc

_NUM_CORES = 2
_NUM_SUBCORES = 16
_NW = _NUM_CORES * _NUM_SUBCORES
_CHUNK = 512


@functools.partial(jax.jit, static_argnums=())
def _gather_flat(indices_flat, table):
    B = indices_flat.shape[0]
    D = table.shape[1]
    assert B % (_NW * _CHUNK) == 0
    b_per_w = B // _NW
    n_chunks = b_per_w // _CHUNK

    mesh = plsc.VectorSubcoreMesh(core_axis_name="c", subcore_axis_name="s")

    @functools.partial(
        pl.kernel,
        mesh=mesh,
        out_type=jax.ShapeDtypeStruct((B, D), jnp.float32),
        scratch_types=[
            pltpu.VMEM((_CHUNK,), jnp.int32),
            pltpu.VMEM((_CHUNK, D), jnp.float32),
            pltpu.SemaphoreType.DMA,
        ],
        compiler_params=pltpu.CompilerParams(use_tc_tiling_on_sc=False),
    )
    def k(idx_hbm, table_hbm, out_hbm, idx_v, rows_v, sem):
        wid = lax.axis_index("s") * _NUM_CORES + lax.axis_index("c")
        base = wid * b_per_w

        def body(i, carry):
            off = base + i * _CHUNK
            pltpu.sync_copy(idx_hbm.at[pl.ds(off, _CHUNK)], idx_v)
            pltpu.async_copy(table_hbm.at[idx_v], rows_v, sem).wait()
            pltpu.sync_copy(rows_v, out_hbm.at[pl.ds(off, _CHUNK)])
            return carry

        lax.fori_loop(0, n_chunks, body, 0)

    return k(indices_flat, table)


def kernel(indices, table):
    shape = indices.shape
    flat = indices.reshape(-1).astype(jnp.int32)
    out = _gather_flat(flat, table)
    return out.reshape(*shape, table.shape[1])


# speedup vs baseline: 1.0450x; 1.0450x over previous
import functools

import jax
import jax.numpy as jnp
from jax import lax
from jax.experimental import pallas as pl
from jax.experimental.pallas import tpu as pltpu
from jax.experimental.pallas import tpu_sc as plsc

_NUM_CORES = 2
_NUM_SUBCORES = 16
_NW = _NUM_CORES * _NUM_SUBCORES
_CHUNK = 512
_NBUF = 2


def _gather_flat(indices_2d, table):
    n_rows, C = indices_2d.shape
    D = table.shape[1]
    B = n_rows * C
    assert n_rows % _NW == 0
    n_chunks = n_rows // _NW
    b_per_w = n_chunks * C

    mesh = plsc.VectorSubcoreMesh(core_axis_name="c", subcore_axis_name="s")

    @functools.partial(
        pl.kernel,
        mesh=mesh,
        out_type=jax.ShapeDtypeStruct((B, D), jnp.float32),
        scratch_types=[
            pltpu.VMEM((n_chunks, C), jnp.int32),
            pltpu.VMEM((_NBUF, C, D), jnp.float32),
            pltpu.SemaphoreType.DMA,
            pltpu.SemaphoreType.DMA,
            pltpu.SemaphoreType.DMA,
            pltpu.SemaphoreType.DMA,
        ],
        compiler_params=pltpu.CompilerParams(use_tc_tiling_on_sc=False),
    )
    def k(idx_hbm, table_hbm, out_hbm, idx_v, rows_v, sg0, sg1, ss0, ss1):
        wid = lax.axis_index("s") * _NUM_CORES + lax.axis_index("c")
        base = wid * b_per_w

        pltpu.sync_copy(idx_hbm.at[pl.ds(wid * n_chunks, n_chunks)], idx_v)

        sg = (sg0, sg1)
        ss = (ss0, ss1)

        def start_gather(i, b):
            pltpu.async_copy(table_hbm.at[idx_v.at[i]], rows_v.at[b], sg[b])

        def wait_gather(i, b):
            pltpu.make_async_copy(
                table_hbm.at[idx_v.at[i]], rows_v.at[b], sg[b]
            ).wait()

        def start_store(i, b):
            pltpu.async_copy(
                rows_v.at[b], out_hbm.at[pl.ds(base + i * C, C)], ss[b]
            )

        def wait_store(i, b):
            pltpu.make_async_copy(
                rows_v.at[b], out_hbm.at[pl.ds(base + i * C, C)], ss[b]
            ).wait()

        start_gather(0, 0)
        wait_gather(0, 0)
        start_store(0, 0)
        start_gather(1, 1)
        wait_gather(1, 1)
        start_store(1, 1)

        def body(j, carry):
            for b in range(_NBUF):
                i = 2 + j * _NBUF + b
                wait_store(i - _NBUF, b)
                start_gather(i, b)
                wait_gather(i, b)
                start_store(i, b)
            return carry

        lax.fori_loop(0, (n_chunks - 2) // _NBUF, body, 0)

        wait_store(n_chunks - 2, 0)
        wait_store(n_chunks - 1, 1)

    return k(indices_2d, table)


def kernel(indices, table):
    shape = indices.shape
    flat = indices.reshape(-1, _CHUNK).astype(jnp.int32)
    out = _gather_flat(flat, table)
    return out.reshape(*shape, table.shape[1])
